# Initial kernel scaffold; baseline (speedup 1.0000x reference)
import jax
import jax.numpy as jnp
from jax.experimental import pallas as pl


def kernel(embeddings, target):
    raise NotImplementedError("write your pallas kernel here")


# speedup vs baseline: 1033.8471x; 1033.8471x over previous
import jax
import jax.numpy as jnp
from jax.experimental import pallas as pl

MARGIN = 1.0
EPS = 1e-6


def _loss_kernel(emb_ref, tgt_ref, out_ref):
    e = emb_ref[...]
    t = tgt_ref[...]
    n, d = e.shape

    g = jax.lax.dot_general(
        e, e, (((1,), (1,)), ((), ())),
        preferred_element_type=jnp.float32,
    )

    sq = jnp.sum(e * e, axis=1, keepdims=True)
    s = jnp.sum(e, axis=1, keepdims=True)

    d2 = sq + sq.T - 2.0 * g + (2.0 * EPS) * (s - s.T) + (d * EPS * EPS)
    d2 = jnp.maximum(d2, 0.0)
    dist = jnp.sqrt(d2)

    pos_mask = t == t.T
    neg = jnp.maximum(MARGIN - dist, 0.0)
    loss = jnp.where(pos_mask, dist * dist, neg * neg)

    row = jax.lax.broadcasted_iota(jnp.int32, (n, n), 0)
    col = jax.lax.broadcasted_iota(jnp.int32, (n, n), 1)
    loss = jnp.where(col > row, loss, 0.0)

    out_ref[...] = jnp.sum(loss, keepdims=True)


def kernel(embeddings, target):
    n = target.shape[0]
    tgt2d = target.astype(jnp.int32).reshape(n, 1)
    loss_sum = pl.pallas_call(
        _loss_kernel,
        out_shape=jax.ShapeDtypeStruct((1, 1), jnp.float32),
    )(embeddings, tgt2d)
    n_pairs = jnp.asarray(n * (n - 1) // 2, dtype=jnp.int32)
    return (loss_sum[0, 0], n_pairs)


# speedup vs baseline: 1136.0160x; 1.0988x over previous
import jax
import jax.numpy as jnp
from jax.experimental import pallas as pl

MARGIN = 1.0
EPS = 1e-6


def _loss_kernel(emb_ref, tgt_ref, out_ref):
    e = emb_ref[...]
    t = tgt_ref[...]
    n, d = e.shape

    g = jax.lax.dot_general(
        e, e, (((1,), (1,)), ((), ())),
        preferred_element_type=jnp.float32,
    )

    sq = jnp.sum(e * e, axis=1, keepdims=True)
    s = jnp.sum(e, axis=1, keepdims=True)

    half_c = 0.5 * d * EPS * EPS
    u = sq + (2.0 * EPS) * s + half_c
    v = sq - (2.0 * EPS) * s + half_c

    d2 = jnp.maximum(u + v.T - 2.0 * g, 0.0)
    dist = jnp.sqrt(d2)

    pos_mask = t == t.T
    neg = jnp.maximum(MARGIN - dist, 0.0)
    loss = jnp.where(pos_mask, d2, neg * neg)

    row = jax.lax.broadcasted_iota(jnp.int32, (n, n), 0)
    col = jax.lax.broadcasted_iota(jnp.int32, (n, n), 1)
    loss = jnp.where(col > row, loss, 0.0)

    out_ref[...] = jnp.sum(loss, keepdims=True)


def kernel(embeddings, target):
    n = target.shape[0]
    tgt2d = target.astype(jnp.int32).reshape(n, 1)
    loss_sum = pl.pallas_call(
        _loss_kernel,
        out_shape=jax.ShapeDtypeStruct((1, 1), jnp.float32),
    )(embeddings, tgt2d)
    n_pairs = jnp.asarray(n * (n - 1) // 2, dtype=jnp.int32)
    return (loss_sum[0, 0], n_pairs)


# speedup vs baseline: 1252.4147x; 1.1025x over previous
import jax
import jax.numpy as jnp
from jax.experimental import pallas as pl

MARGIN = 1.0
EPS = 1e-6


def _loss_kernel(emb_ref, tgt_ref, out_ref):
    e = emb_ref[...]
    t = tgt_ref[...].reshape(-1, 1)
    n, d = e.shape

    g = jax.lax.dot_general(
        e, e, (((1,), (1,)), ((), ())),
        preferred_element_type=jnp.float32,
    )

    sq = jnp.sum(e * e, axis=1, keepdims=True)
    s = jnp.sum(e, axis=1, keepdims=True)

    half_c = 0.5 * d * EPS * EPS
    u = sq + (2.0 * EPS) * s + half_c
    v = sq - (2.0 * EPS) * s + half_c

    d2 = jnp.maximum(u + v.T - 2.0 * g, 0.0)
    dist = jnp.sqrt(d2)

    pos_mask = t == t.T
    neg = jnp.maximum(MARGIN - dist, 0.0)
    loss = jnp.where(pos_mask, d2, neg * neg)

    row = jax.lax.broadcasted_iota(jnp.int32, (n, n), 0)
    col = jax.lax.broadcasted_iota(jnp.int32, (n, n), 1)
    loss = jnp.where(col > row, loss, 0.0)

    out_ref[...] = jnp.sum(loss, keepdims=True)


def kernel(embeddings, target):
    n = target.shape[0]
    loss_sum = pl.pallas_call(
        _loss_kernel,
        out_shape=jax.ShapeDtypeStruct((1, 1), jnp.float32),
    )(embeddings, target)
    n_pairs = jnp.asarray(n * (n - 1) // 2, dtype=jnp.int32)
    return (loss_sum[0, 0], n_pairs)


# speedup vs baseline: 1693.0474x; 1.3518x over previous
import jax
import jax.numpy as jnp
from jax.experimental import pallas as pl

MARGIN = 1.0
EPS = 1e-6


def _loss_kernel(emb_ref, tgt_ref, out_ref):
    e = emb_ref[...]
    t = tgt_ref[...].reshape(-1, 1)
    n, d = e.shape

    sq = jnp.sum(e * e, axis=1, keepdims=True)
    s = jnp.sum(e, axis=1, keepdims=True)

    half_c = 0.5 * d * EPS * EPS
    u = sq + (2.0 * EPS) * s + half_c
    v = sq - (2.0 * EPS) * s + half_c
    ones = jnp.ones((n, 1), jnp.float32)
    lhs = jnp.concatenate([e * -2.0, u, ones], axis=1)
    rhs = jnp.concatenate([e, ones, v], axis=1)

    d2 = jax.lax.dot_general(
        lhs, rhs, (((1,), (1,)), ((), ())),
        preferred_element_type=jnp.float32,
    )
    d2 = jnp.maximum(d2, 0.0)
    dist = jnp.sqrt(d2)

    pos_mask = t == t.T
    neg = jnp.maximum(MARGIN - dist, 0.0)
    loss = jnp.where(pos_mask, d2, neg * neg)

    row = jax.lax.broadcasted_iota(jnp.int32, (n, n), 0)
    col = jax.lax.broadcasted_iota(jnp.int32, (n, n), 1)
    loss = jnp.where(col > row, loss, 0.0)

    out_ref[...] = jnp.sum(loss, keepdims=True)


def kernel(embeddings, target):
    n = target.shape[0]
    loss_sum = pl.pallas_call(
        _loss_kernel,
        out_shape=jax.ShapeDtypeStruct((1, 1), jnp.float32),
    )(embeddings, target)
    n_pairs = jnp.asarray(n * (n - 1) // 2, dtype=jnp.int32)
    return (loss_sum[0, 0], n_pairs)


# speedup vs baseline: 1753.8749x; 1.0359x over previous
import jax
import jax.numpy as jnp
from jax.experimental import pallas as pl

MARGIN = 1.0
EPS = 1e-6


def _loss_kernel(emb_ref, tgt_ref, out_ref):
    e = emb_ref[...]
    t = tgt_ref[...].reshape(-1, 1)
    n, d = e.shape

    sq = jnp.sum(e * e, axis=1, keepdims=True)
    s = jnp.sum(e, axis=1, keepdims=True)

    half_c = 0.5 * d * EPS * EPS
    u = sq + (2.0 * EPS) * s + half_c
    v = sq - (2.0 * EPS) * s + half_c
    ones = jnp.ones((n, 1), jnp.float32)
    lhs = jnp.concatenate([e * -2.0, u, ones], axis=1)
    rhs = jnp.concatenate([e, ones, v], axis=1)

    d2 = jax.lax.dot_general(
        lhs, rhs, (((1,), (1,)), ((), ())),
        preferred_element_type=jnp.float32,
    )
    d2 = jnp.maximum(d2, 0.0)
    d2c = jnp.maximum(d2, 1e-12)
    dist = d2c * jax.lax.rsqrt(d2c)

    pos_mask = t == t.T
    neg = jnp.maximum(MARGIN - dist, 0.0)
    loss = jnp.where(pos_mask, d2, neg * neg)

    row = jax.lax.broadcasted_iota(jnp.int32, (n, n), 0)
    col = jax.lax.broadcasted_iota(jnp.int32, (n, n), 1)
    loss = jnp.where(col > row, loss, 0.0)

    out_ref[...] = jnp.sum(loss, keepdims=True)


def kernel(embeddings, target):
    n = target.shape[0]
    loss_sum = pl.pallas_call(
        _loss_kernel,
        out_shape=jax.ShapeDtypeStruct((1, 1), jnp.float32),
    )(embeddings, target)
    n_pairs = jnp.asarray(n * (n - 1) // 2, dtype=jnp.int32)
    return (loss_sum[0, 0], n_pairs)
